# baseline (speedup 1.0000x reference)
import functools

import jax
import jax.numpy as jnp
from jax import lax
from jax.experimental import pallas as pl
from jax.experimental.pallas import tpu as pltpu
from jax.experimental.pallas import tpu_sc as plsc

N = 10000
D = 128
E = 160000
NC, NS, L = 2, 16, 16
CHUNK = 96
CPT = 105
EPT = NS * CPT * CHUNK
R = 10112
RPT = R // NS
MROWS = R // NS


def _sc_agg(xcat, src_idx, dst_idx, init):
    mesh = plsc.VectorSubcoreMesh(core_axis_name="c", subcore_axis_name="s")

    @functools.partial(
        pl.kernel,
        mesh=mesh,
        out_type=jax.ShapeDtypeStruct((NC * R, D), jnp.float32),
        scratch_types=[
            pltpu.VMEM((CPT * CHUNK,), jnp.int32),
            pltpu.VMEM((3, CHUNK), jnp.int32),
            pltpu.VMEM((CHUNK, D), jnp.float32),
            pltpu.VMEM((CHUNK, D), jnp.float32),
            pltpu.VMEM((CHUNK, D), jnp.float32),
            pltpu.VMEM_SHARED((R, D), jnp.float32),
            pltpu.SemaphoreType.DMA,
            pltpu.SemaphoreType.DMA,
            pltpu.SemaphoreType.DMA,
            pltpu.SemaphoreType.DMA,
            pltpu.SemaphoreType.DMA,
            pltpu.SemaphoreType.DMA,
        ],
    )
    def k(xcat_hbm, src_hbm, dst_hbm, init_hbm, out_hbm,
          src_v, dring, rows0, rows1, rows2, accum,
          sg0, sg1, sg2, sd0, sd1, sd2):
        c = lax.axis_index("c")
        s = lax.axis_index("s")
        w = c * NS + s

        pltpu.sync_copy(src_hbm.at[w], src_v)
        pltpu.sync_copy(init_hbm.at[pl.ds(c * R + s * RPT, RPT)],
                        accum.at[pl.ds(s * RPT, RPT)])
        plsc.subcore_barrier()

        def gidx(j):
            return src_v.at[pl.ds(j * CHUNK, CHUNK)]

        bufs = ((rows0, sg0, sd0), (rows1, sg1, sd1), (rows2, sg2, sd2))
        for b, (rows, sg, sd) in enumerate(bufs):
            pltpu.async_copy(xcat_hbm.at[gidx(b)], rows, sg)
            pltpu.async_copy(dst_hbm.at[w * CPT + b], dring.at[b], sd)

        def body(g, carry):
            j = 3 * g
            for b, (rows, sg, sd) in enumerate(bufs):
                pltpu.make_async_copy(xcat_hbm.at[gidx(j + b)], rows, sg).wait()
                pltpu.make_async_copy(dst_hbm.at[w * CPT + j + b],
                                      dring.at[b], sd).wait()
                pltpu.sync_copy(rows, accum.at[dring.at[b]], add=True)

                @pl.when(j + b + 3 < CPT)
                def _():
                    pltpu.async_copy(xcat_hbm.at[gidx(j + b + 3)], rows, sg)
                    pltpu.async_copy(dst_hbm.at[w * CPT + j + b + 3],
                                     dring.at[b], sd)

            return carry

        lax.fori_loop(0, CPT // 3, body, 0)
        plsc.subcore_barrier()

        pltpu.sync_copy(accum.at[pl.ds(s * RPT, RPT)],
                        out_hbm.at[pl.ds(c * R + s * RPT, RPT)])

    return k(xcat, src_idx, dst_idx, init)


def _tc_mlp_body(x_ref, w1_ref, b1_ref, w2_ref, b2_ref, o_ref):
    h = jnp.dot(x_ref[...], w1_ref[0], preferred_element_type=jnp.float32)
    h = jnp.maximum(h + b1_ref[0], 0.0)
    y = jnp.dot(h, w2_ref[0], preferred_element_type=jnp.float32)
    o_ref[...] = jnp.maximum(y + b2_ref[0], 0.0)


def _tc_mlp(xin, w1s, b1s, w2s, b2s):
    grid = (NC * R // MROWS,)
    return pl.pallas_call(
        _tc_mlp_body,
        grid=grid,
        in_specs=[
            pl.BlockSpec((MROWS, D), lambda i: (i, 0)),
            pl.BlockSpec((1, D, D), lambda i: (i // (R // MROWS), 0, 0)),
            pl.BlockSpec((1, 1, D), lambda i: (i // (R // MROWS), 0, 0)),
            pl.BlockSpec((1, D, D), lambda i: (i // (R // MROWS), 0, 0)),
            pl.BlockSpec((1, 1, D), lambda i: (i // (R // MROWS), 0, 0)),
        ],
        out_specs=pl.BlockSpec((MROWS, D), lambda i: (i, 0)),
        out_shape=jax.ShapeDtypeStruct((NC * R, D), jnp.float32),
    )(xin, w1s, b1s, w2s, b2s)


def _fold_bn(W1, b1, g1, be1, rm1, rv1, W2, b2, g2, be2, rm2, rv2):
    s1 = g1 * lax.rsqrt(rv1 + 1e-5)
    s2 = g2 * lax.rsqrt(rv2 + 1e-5)
    return (W1 * s1[None, :], (b1 - rm1) * s1 + be1,
            W2 * s2[None, :], (b2 - rm2) * s2 + be2)


def kernel(x_op, x_mach, ei_om, ei_mo,
           W1_op, b1_op, g1_op, be1_op, rm1_op, rv1_op,
           W2_op, b2_op, g2_op, be2_op, rm2_op, rv2_op,
           W1_mach, b1_mach, g1_mach, be1_mach, rm1_mach, rv1_mach,
           W2_mach, b2_mach, g2_mach, be2_mach, rm2_mach, rv2_mach,
           eps_om, eps_mo):
    pad = EPT - E
    zpad_i = jnp.zeros((pad,), jnp.int32)
    dpad_i = jnp.full((pad,), N, jnp.int32)

    xcat = jnp.concatenate([x_op, x_mach], axis=0)
    src_all = jnp.concatenate(
        [ei_om[0], zpad_i, ei_mo[0] + N, zpad_i]).reshape(NC * NS, CPT * CHUNK)
    dst_all = jnp.concatenate(
        [ei_om[1], dpad_i, ei_mo[1], dpad_i]).reshape(NC * NS * CPT, CHUNK)

    init = jnp.zeros((NC, R, D), jnp.float32)
    init = init.at[0, :N].set((1.0 + eps_om) * x_mach)
    init = init.at[1, :N].set((1.0 + eps_mo) * x_op)
    init = init.reshape(NC * R, D)

    agg = _sc_agg(xcat, src_all, dst_all, init)

    w1f_op, b1f_op, w2f_op, b2f_op = _fold_bn(
        W1_op, b1_op, g1_op, be1_op, rm1_op, rv1_op,
        W2_op, b2_op, g2_op, be2_op, rm2_op, rv2_op)
    w1f_m, b1f_m, w2f_m, b2f_m = _fold_bn(
        W1_mach, b1_mach, g1_mach, be1_mach, rm1_mach, rv1_mach,
        W2_mach, b2_mach, g2_mach, be2_mach, rm2_mach, rv2_mach)

    w1s = jnp.stack([w1f_op, w1f_m])
    b1s = jnp.stack([b1f_op, b1f_m])[:, None, :]
    w2s = jnp.stack([w2f_op, w2f_m])
    b2s = jnp.stack([b2f_op, b2f_m])[:, None, :]

    y = _tc_mlp(agg, w1s, b1s, w2s, b2s)
    out_mach = y[:N]
    out_op = y[R:R + N]
    return (out_op, out_mach)


# speedup vs baseline: 1.0000x; 1.0000x over previous
import functools

import jax
import jax.numpy as jnp
from jax import lax
from jax.experimental import pallas as pl
from jax.experimental.pallas import tpu as pltpu
from jax.experimental.pallas import tpu_sc as plsc

N = 10000
D = 128
E = 160000
NC, NS, L = 2, 16, 16
NBUF = 4
CHUNK = 72
CPT = 140
EPT = NS * CPT * CHUNK
R = 10112
RPT = R // NS
MROWS = R // NS


def _sc_agg(xcat, src_idx, dst_idx, init):
    mesh = plsc.VectorSubcoreMesh(core_axis_name="c", subcore_axis_name="s")

    @functools.partial(
        pl.kernel,
        mesh=mesh,
        out_type=jax.ShapeDtypeStruct((NC * R, D), jnp.float32),
        scratch_types=(
            [pltpu.VMEM((CPT * CHUNK,), jnp.int32),
             pltpu.VMEM((NBUF, CHUNK), jnp.int32)]
            + [pltpu.VMEM((CHUNK, D), jnp.float32)] * NBUF
            + [pltpu.VMEM_SHARED((R, D), jnp.float32)]
            + [pltpu.SemaphoreType.DMA] * (2 * NBUF)
        ),
    )
    def k(xcat_hbm, src_hbm, dst_hbm, init_hbm, out_hbm,
          src_v, dring, *rest):
        rows_l = rest[:NBUF]
        accum = rest[NBUF]
        sg_l = rest[NBUF + 1:NBUF + 1 + NBUF]
        sd_l = rest[NBUF + 1 + NBUF:]
        c = lax.axis_index("c")
        s = lax.axis_index("s")
        w = c * NS + s

        pltpu.sync_copy(src_hbm.at[w], src_v)
        pltpu.sync_copy(init_hbm.at[pl.ds(c * R + s * RPT, RPT)],
                        accum.at[pl.ds(s * RPT, RPT)])
        plsc.subcore_barrier()

        def gidx(j):
            return src_v.at[pl.ds(j * CHUNK, CHUNK)]

        bufs = tuple(zip(rows_l, sg_l, sd_l))
        for b, (rows, sg, sd) in enumerate(bufs):
            pltpu.async_copy(xcat_hbm.at[gidx(b)], rows, sg)
            pltpu.async_copy(dst_hbm.at[w * CPT + b], dring.at[b], sd)

        def body(g, carry):
            j = NBUF * g
            for b, (rows, sg, sd) in enumerate(bufs):
                pltpu.make_async_copy(xcat_hbm.at[gidx(j + b)], rows, sg).wait()
                pltpu.make_async_copy(dst_hbm.at[w * CPT + j + b],
                                      dring.at[b], sd).wait()
                pltpu.sync_copy(rows, accum.at[dring.at[b]], add=True)

                @pl.when(j + b + NBUF < CPT)
                def _():
                    pltpu.async_copy(xcat_hbm.at[gidx(j + b + NBUF)], rows, sg)
                    pltpu.async_copy(dst_hbm.at[w * CPT + j + b + NBUF],
                                     dring.at[b], sd)

            return carry

        lax.fori_loop(0, CPT // NBUF, body, 0)
        plsc.subcore_barrier()

        pltpu.sync_copy(accum.at[pl.ds(s * RPT, RPT)],
                        out_hbm.at[pl.ds(c * R + s * RPT, RPT)])

    return k(xcat, src_idx, dst_idx, init)


def _tc_mlp_body(x_ref, w1_ref, b1_ref, w2_ref, b2_ref, o_ref):
    h = jnp.dot(x_ref[...], w1_ref[0], preferred_element_type=jnp.float32)
    h = jnp.maximum(h + b1_ref[0], 0.0)
    y = jnp.dot(h, w2_ref[0], preferred_element_type=jnp.float32)
    o_ref[...] = jnp.maximum(y + b2_ref[0], 0.0)


def _tc_mlp(xin, w1s, b1s, w2s, b2s):
    grid = (NC * R // MROWS,)
    return pl.pallas_call(
        _tc_mlp_body,
        grid=grid,
        in_specs=[
            pl.BlockSpec((MROWS, D), lambda i: (i, 0)),
            pl.BlockSpec((1, D, D), lambda i: (i // (R // MROWS), 0, 0)),
            pl.BlockSpec((1, 1, D), lambda i: (i // (R // MROWS), 0, 0)),
            pl.BlockSpec((1, D, D), lambda i: (i // (R // MROWS), 0, 0)),
            pl.BlockSpec((1, 1, D), lambda i: (i // (R // MROWS), 0, 0)),
        ],
        out_specs=pl.BlockSpec((MROWS, D), lambda i: (i, 0)),
        out_shape=jax.ShapeDtypeStruct((NC * R, D), jnp.float32),
    )(xin, w1s, b1s, w2s, b2s)


def _fold_bn(W1, b1, g1, be1, rm1, rv1, W2, b2, g2, be2, rm2, rv2):
    s1 = g1 * lax.rsqrt(rv1 + 1e-5)
    s2 = g2 * lax.rsqrt(rv2 + 1e-5)
    return (W1 * s1[None, :], (b1 - rm1) * s1 + be1,
            W2 * s2[None, :], (b2 - rm2) * s2 + be2)


def kernel(x_op, x_mach, ei_om, ei_mo,
           W1_op, b1_op, g1_op, be1_op, rm1_op, rv1_op,
           W2_op, b2_op, g2_op, be2_op, rm2_op, rv2_op,
           W1_mach, b1_mach, g1_mach, be1_mach, rm1_mach, rv1_mach,
           W2_mach, b2_mach, g2_mach, be2_mach, rm2_mach, rv2_mach,
           eps_om, eps_mo):
    pad = EPT - E
    zpad_i = jnp.zeros((pad,), jnp.int32)
    dpad_i = jnp.full((pad,), N, jnp.int32)

    xcat = jnp.concatenate([x_op, x_mach], axis=0)
    src_all = jnp.concatenate(
        [ei_om[0], zpad_i, ei_mo[0] + N, zpad_i]).reshape(NC * NS, CPT * CHUNK)
    dst_all = jnp.concatenate(
        [ei_om[1], dpad_i, ei_mo[1], dpad_i]).reshape(NC * NS * CPT, CHUNK)

    init = jnp.zeros((NC, R, D), jnp.float32)
    init = init.at[0, :N].set((1.0 + eps_om) * x_mach)
    init = init.at[1, :N].set((1.0 + eps_mo) * x_op)
    init = init.reshape(NC * R, D)

    agg = _sc_agg(xcat, src_all, dst_all, init)

    w1f_op, b1f_op, w2f_op, b2f_op = _fold_bn(
        W1_op, b1_op, g1_op, be1_op, rm1_op, rv1_op,
        W2_op, b2_op, g2_op, be2_op, rm2_op, rv2_op)
    w1f_m, b1f_m, w2f_m, b2f_m = _fold_bn(
        W1_mach, b1_mach, g1_mach, be1_mach, rm1_mach, rv1_mach,
        W2_mach, b2_mach, g2_mach, be2_mach, rm2_mach, rv2_mach)

    w1s = jnp.stack([w1f_op, w1f_m])
    b1s = jnp.stack([b1f_op, b1f_m])[:, None, :]
    w2s = jnp.stack([w2f_op, w2f_m])
    b2s = jnp.stack([b2f_op, b2f_m])[:, None, :]

    y = _tc_mlp(agg, w1s, b1s, w2s, b2s)
    out_mach = y[:N]
    out_op = y[R:R + N]
    return (out_op, out_mach)


# speedup vs baseline: 1.0838x; 1.0837x over previous
import functools

import jax
import jax.numpy as jnp
from jax import lax
from jax.experimental import pallas as pl
from jax.experimental.pallas import tpu as pltpu
from jax.experimental.pallas import tpu_sc as plsc

N = 10000
D = 128
E = 160000
NC, NS, L = 2, 16, 16
NBUF = 3
CHUNK = 96
CPT = 105
EPT = NS * CPT * CHUNK
RACC = N + 8
RPT = 632
LASTR = N - (NS - 1) * RPT
MROWS = 400


def _sc_agg(xcat, src_idx, dst_idx, x_op, x_mach):
    mesh = plsc.VectorSubcoreMesh(core_axis_name="c", subcore_axis_name="s")

    @functools.partial(
        pl.kernel,
        mesh=mesh,
        out_type=jax.ShapeDtypeStruct((NC * N, D), jnp.float32),
        scratch_types=(
            [pltpu.VMEM((CPT * CHUNK,), jnp.int32),
             pltpu.VMEM((NBUF, CHUNK), jnp.int32)]
            + [pltpu.VMEM((CHUNK, D), jnp.float32)] * NBUF
            + [pltpu.VMEM_SHARED((RACC, D), jnp.float32)]
            + [pltpu.SemaphoreType.DMA] * (2 * NBUF)
        ),
    )
    def k(xcat_hbm, src_hbm, dst_hbm, xop_hbm, xmach_hbm, out_hbm,
          src_v, dring, *rest):
        rows_l = rest[:NBUF]
        accum = rest[NBUF]
        sg_l = rest[NBUF + 1:NBUF + 1 + NBUF]
        sd_l = rest[NBUF + 1 + NBUF:]
        c = lax.axis_index("c")
        s = lax.axis_index("s")
        w = c * NS + s

        pltpu.sync_copy(src_hbm.at[w], src_v)

        for cc, xd in ((0, xmach_hbm), (1, xop_hbm)):
            @pl.when((c == cc) & (s < NS - 1))
            def _(xd=xd):
                pltpu.sync_copy(xd.at[pl.ds(s * RPT, RPT)],
                                accum.at[pl.ds(s * RPT, RPT)])

            @pl.when((c == cc) & (s == NS - 1))
            def _(xd=xd):
                pltpu.sync_copy(xd.at[pl.ds((NS - 1) * RPT, LASTR)],
                                accum.at[pl.ds((NS - 1) * RPT, LASTR)])

        plsc.subcore_barrier()

        def gidx(j):
            return src_v.at[pl.ds(j * CHUNK, CHUNK)]

        bufs = tuple(zip(rows_l, sg_l, sd_l))
        for b, (rows, sg, sd) in enumerate(bufs):
            pltpu.async_copy(xcat_hbm.at[gidx(b)], rows, sg)
            pltpu.async_copy(dst_hbm.at[w * CPT + b], dring.at[b], sd)

        def body(g, carry):
            j = NBUF * g
            for b, (rows, sg, sd) in enumerate(bufs):
                pltpu.make_async_copy(xcat_hbm.at[gidx(j + b)], rows, sg).wait()
                pltpu.make_async_copy(dst_hbm.at[w * CPT + j + b],
                                      dring.at[b], sd).wait()
                pltpu.sync_copy(rows, accum.at[dring.at[b]], add=True)

                @pl.when(j + b + NBUF < CPT)
                def _():
                    pltpu.async_copy(xcat_hbm.at[gidx(j + b + NBUF)], rows, sg)
                    pltpu.async_copy(dst_hbm.at[w * CPT + j + b + NBUF],
                                     dring.at[b], sd)

            return carry

        lax.fori_loop(0, CPT // NBUF, body, 0)
        plsc.subcore_barrier()

        @pl.when(s < NS - 1)
        def _():
            pltpu.sync_copy(accum.at[pl.ds(s * RPT, RPT)],
                            out_hbm.at[pl.ds(c * N + s * RPT, RPT)])

        @pl.when(s == NS - 1)
        def _():
            pltpu.sync_copy(accum.at[pl.ds((NS - 1) * RPT, LASTR)],
                            out_hbm.at[pl.ds(c * N + (NS - 1) * RPT, LASTR)])

    return k(xcat, src_idx, dst_idx, x_op, x_mach)


def _tc_mlp_body(agg0_ref, agg1_ref, xm_ref, xo_ref,
                 w1_ref, b1_ref, w2_ref, b2_ref, eps_ref,
                 o0_ref, o1_ref):
    def mlp(xin, t):
        h = jnp.dot(xin, w1_ref[t], preferred_element_type=jnp.float32)
        h = jnp.maximum(h + b1_ref[t], 0.0)
        y = jnp.dot(h, w2_ref[t], preferred_element_type=jnp.float32)
        return jnp.maximum(y + b2_ref[t], 0.0)

    o0_ref[...] = mlp(agg0_ref[...] + eps_ref[0] * xm_ref[...], 0)
    o1_ref[...] = mlp(agg1_ref[...] + eps_ref[1] * xo_ref[...], 1)


def _tc_mlp(agg, x_mach, x_op, w1s, b1s, w2s, b2s, epss):
    nb = N // MROWS
    out = pl.pallas_call(
        _tc_mlp_body,
        grid=(nb,),
        in_specs=[
            pl.BlockSpec((MROWS, D), lambda i: (i, 0)),
            pl.BlockSpec((MROWS, D), lambda i, _nb=nb: (i + _nb, 0)),
            pl.BlockSpec((MROWS, D), lambda i: (i, 0)),
            pl.BlockSpec((MROWS, D), lambda i: (i, 0)),
            pl.BlockSpec((NC, D, D), lambda i: (0, 0, 0)),
            pl.BlockSpec((NC, 1, D), lambda i: (0, 0, 0)),
            pl.BlockSpec((NC, D, D), lambda i: (0, 0, 0)),
            pl.BlockSpec((NC, 1, D), lambda i: (0, 0, 0)),
            pl.BlockSpec(memory_space=pltpu.SMEM),
        ],
        out_specs=[
            pl.BlockSpec((MROWS, D), lambda i: (i, 0)),
            pl.BlockSpec((MROWS, D), lambda i: (i, 0)),
        ],
        out_shape=[
            jax.ShapeDtypeStruct((N, D), jnp.float32),
            jax.ShapeDtypeStruct((N, D), jnp.float32),
        ],
    )(agg, agg, x_mach, x_op, w1s, b1s, w2s, b2s, epss)
    return out


def _fold_bn(W1, b1, g1, be1, rm1, rv1, W2, b2, g2, be2, rm2, rv2):
    s1 = g1 * lax.rsqrt(rv1 + 1e-5)
    s2 = g2 * lax.rsqrt(rv2 + 1e-5)
    return (W1 * s1[None, :], (b1 - rm1) * s1 + be1,
            W2 * s2[None, :], (b2 - rm2) * s2 + be2)


def kernel(x_op, x_mach, ei_om, ei_mo,
           W1_op, b1_op, g1_op, be1_op, rm1_op, rv1_op,
           W2_op, b2_op, g2_op, be2_op, rm2_op, rv2_op,
           W1_mach, b1_mach, g1_mach, be1_mach, rm1_mach, rv1_mach,
           W2_mach, b2_mach, g2_mach, be2_mach, rm2_mach, rv2_mach,
           eps_om, eps_mo):
    pad = EPT - E
    zpad_i = jnp.zeros((pad,), jnp.int32)
    dpad_i = jnp.full((pad,), N, jnp.int32)

    xcat = jnp.concatenate([x_op, x_mach], axis=0)
    src_all = jnp.concatenate(
        [ei_om[0], zpad_i, ei_mo[0] + N, zpad_i]).reshape(NC * NS, CPT * CHUNK)
    dst_all = jnp.concatenate(
        [ei_om[1], dpad_i, ei_mo[1], dpad_i]).reshape(NC * NS * CPT, CHUNK)

    agg = _sc_agg(xcat, src_all, dst_all, x_op, x_mach)

    w1f_op, b1f_op, w2f_op, b2f_op = _fold_bn(
        W1_op, b1_op, g1_op, be1_op, rm1_op, rv1_op,
        W2_op, b2_op, g2_op, be2_op, rm2_op, rv2_op)
    w1f_m, b1f_m, w2f_m, b2f_m = _fold_bn(
        W1_mach, b1_mach, g1_mach, be1_mach, rm1_mach, rv1_mach,
        W2_mach, b2_mach, g2_mach, be2_mach, rm2_mach, rv2_mach)

    w1s = jnp.stack([w1f_op, w1f_m])
    b1s = jnp.stack([b1f_op, b1f_m])[:, None, :]
    w2s = jnp.stack([w2f_op, w2f_m])
    b2s = jnp.stack([b2f_op, b2f_m])[:, None, :]
    epss = jnp.stack([eps_om, eps_mo])

    out_mach, out_op = _tc_mlp(agg, x_mach, x_op, w1s, b1s, w2s, b2s, epss)
    return (out_op, out_mach)


# speedup vs baseline: 1.0970x; 1.0122x over previous
import functools

import jax
import jax.numpy as jnp
from jax import lax
from jax.experimental import pallas as pl
from jax.experimental.pallas import tpu as pltpu
from jax.experimental.pallas import tpu_sc as plsc

N = 10000
D = 128
E = 160000
NC, NS, L = 2, 16, 16
NBUF = 3
CHUNK = 96
CPT = 105
EPT = NS * CPT * CHUNK
RACC = N + 8
RPT = 632
LASTR = N - (NS - 1) * RPT
MROWS = 400


def _sc_agg(xcat, src_idx, dst_idx, x_op, x_mach):
    mesh = plsc.VectorSubcoreMesh(core_axis_name="c", subcore_axis_name="s")

    @functools.partial(
        pl.kernel,
        mesh=mesh,
        out_type=jax.ShapeDtypeStruct((NC * N, D), jnp.float32),
        scratch_types=(
            [pltpu.VMEM((CPT * CHUNK,), jnp.int32),
             pltpu.VMEM((NBUF, CHUNK), jnp.int32)]
            + [pltpu.VMEM((CHUNK, D), jnp.float32)] * NBUF
            + [pltpu.VMEM_SHARED((RACC, D), jnp.float32)]
            + [pltpu.SemaphoreType.DMA] * (2 * NBUF + 2)
        ),
    )
    def k(xcat_hbm, src_hbm, dst_hbm, xop_hbm, xmach_hbm, out_hbm,
          src_v, dring, *rest):
        rows_l = rest[:NBUF]
        accum = rest[NBUF]
        sg_l = rest[NBUF + 1:NBUF + 1 + NBUF]
        sd_l = rest[NBUF + 1 + NBUF:NBUF + 1 + 2 * NBUF]
        s_stage, s_init = rest[NBUF + 1 + 2 * NBUF:]
        c = lax.axis_index("c")
        s = lax.axis_index("s")
        w = c * NS + s

        stage_cp = pltpu.async_copy(src_hbm.at[w], src_v, s_stage)

        for cc, xd in ((0, xmach_hbm), (1, xop_hbm)):
            @pl.when((c == cc) & (s < NS - 1))
            def _(xd=xd):
                pltpu.async_copy(xd.at[pl.ds(s * RPT, RPT)],
                                 accum.at[pl.ds(s * RPT, RPT)], s_init)

            @pl.when((c == cc) & (s == NS - 1))
            def _(xd=xd):
                pltpu.async_copy(xd.at[pl.ds((NS - 1) * RPT, LASTR)],
                                 accum.at[pl.ds((NS - 1) * RPT, LASTR)], s_init)

        def gidx(j):
            return src_v.at[pl.ds(j * CHUNK, CHUNK)]

        stage_cp.wait()
        bufs = tuple(zip(rows_l, sg_l, sd_l))
        for b, (rows, sg, sd) in enumerate(bufs):
            pltpu.async_copy(xcat_hbm.at[gidx(b)], rows, sg)
            pltpu.async_copy(dst_hbm.at[w * CPT + b], dring.at[b], sd)

        @pl.when(s < NS - 1)
        def _():
            pltpu.make_async_copy(
                xmach_hbm.at[pl.ds(s * RPT, RPT)],
                accum.at[pl.ds(s * RPT, RPT)], s_init).wait()

        @pl.when(s == NS - 1)
        def _():
            pltpu.make_async_copy(
                xmach_hbm.at[pl.ds((NS - 1) * RPT, LASTR)],
                accum.at[pl.ds((NS - 1) * RPT, LASTR)], s_init).wait()

        plsc.subcore_barrier()

        def body(g, carry):
            j = NBUF * g
            for b, (rows, sg, sd) in enumerate(bufs):
                pltpu.make_async_copy(xcat_hbm.at[gidx(j + b)], rows, sg).wait()
                pltpu.make_async_copy(dst_hbm.at[w * CPT + j + b],
                                      dring.at[b], sd).wait()
                pltpu.sync_copy(rows, accum.at[dring.at[b]], add=True)

                @pl.when(j + b + NBUF < CPT)
                def _():
                    pltpu.async_copy(xcat_hbm.at[gidx(j + b + NBUF)], rows, sg)
                    pltpu.async_copy(dst_hbm.at[w * CPT + j + b + NBUF],
                                     dring.at[b], sd)

            return carry

        lax.fori_loop(0, CPT // NBUF, body, 0)
        plsc.subcore_barrier()

        @pl.when(s < NS - 1)
        def _():
            pltpu.sync_copy(accum.at[pl.ds(s * RPT, RPT)],
                            out_hbm.at[pl.ds(c * N + s * RPT, RPT)])

        @pl.when(s == NS - 1)
        def _():
            pltpu.sync_copy(accum.at[pl.ds((NS - 1) * RPT, LASTR)],
                            out_hbm.at[pl.ds(c * N + (NS - 1) * RPT, LASTR)])

    return k(xcat, src_idx, dst_idx, x_op, x_mach)


def _tc_mlp_body(agg0_ref, agg1_ref, xm_ref, xo_ref,
                 w1_ref, b1_ref, w2_ref, b2_ref, eps_ref,
                 o0_ref, o1_ref):
    def mlp(xin, t):
        h = jnp.dot(xin, w1_ref[t], preferred_element_type=jnp.float32)
        h = jnp.maximum(h + b1_ref[t], 0.0)
        y = jnp.dot(h, w2_ref[t], preferred_element_type=jnp.float32)
        return jnp.maximum(y + b2_ref[t], 0.0)

    o0_ref[...] = mlp(agg0_ref[...] + eps_ref[0] * xm_ref[...], 0)
    o1_ref[...] = mlp(agg1_ref[...] + eps_ref[1] * xo_ref[...], 1)


def _tc_mlp(agg, x_mach, x_op, w1s, b1s, w2s, b2s, epss):
    nb = N // MROWS
    out = pl.pallas_call(
        _tc_mlp_body,
        grid=(nb,),
        in_specs=[
            pl.BlockSpec((MROWS, D), lambda i: (i, 0)),
            pl.BlockSpec((MROWS, D), lambda i, _nb=nb: (i + _nb, 0)),
            pl.BlockSpec((MROWS, D), lambda i: (i, 0)),
            pl.BlockSpec((MROWS, D), lambda i: (i, 0)),
            pl.BlockSpec((NC, D, D), lambda i: (0, 0, 0)),
            pl.BlockSpec((NC, 1, D), lambda i: (0, 0, 0)),
            pl.BlockSpec((NC, D, D), lambda i: (0, 0, 0)),
            pl.BlockSpec((NC, 1, D), lambda i: (0, 0, 0)),
            pl.BlockSpec(memory_space=pltpu.SMEM),
        ],
        out_specs=[
            pl.BlockSpec((MROWS, D), lambda i: (i, 0)),
            pl.BlockSpec((MROWS, D), lambda i: (i, 0)),
        ],
        out_shape=[
            jax.ShapeDtypeStruct((N, D), jnp.float32),
            jax.ShapeDtypeStruct((N, D), jnp.float32),
        ],
    )(agg, agg, x_mach, x_op, w1s, b1s, w2s, b2s, epss)
    return out


def _fold_bn(W1, b1, g1, be1, rm1, rv1, W2, b2, g2, be2, rm2, rv2):
    s1 = g1 * lax.rsqrt(rv1 + 1e-5)
    s2 = g2 * lax.rsqrt(rv2 + 1e-5)
    return (W1 * s1[None, :], (b1 - rm1) * s1 + be1,
            W2 * s2[None, :], (b2 - rm2) * s2 + be2)


def kernel(x_op, x_mach, ei_om, ei_mo,
           W1_op, b1_op, g1_op, be1_op, rm1_op, rv1_op,
           W2_op, b2_op, g2_op, be2_op, rm2_op, rv2_op,
           W1_mach, b1_mach, g1_mach, be1_mach, rm1_mach, rv1_mach,
           W2_mach, b2_mach, g2_mach, be2_mach, rm2_mach, rv2_mach,
           eps_om, eps_mo):
    pad = EPT - E
    zpad_i = jnp.zeros((pad,), jnp.int32)
    dpad_i = jnp.full((pad,), N, jnp.int32)

    xcat = jnp.concatenate([x_op, x_mach], axis=0)
    src_all = jnp.concatenate(
        [ei_om[0], zpad_i, ei_mo[0] + N, zpad_i]).reshape(NC * NS, CPT * CHUNK)
    dst_all = jnp.concatenate(
        [ei_om[1], dpad_i, ei_mo[1], dpad_i]).reshape(NC * NS * CPT, CHUNK)

    agg = _sc_agg(xcat, src_all, dst_all, x_op, x_mach)

    w1f_op, b1f_op, w2f_op, b2f_op = _fold_bn(
        W1_op, b1_op, g1_op, be1_op, rm1_op, rv1_op,
        W2_op, b2_op, g2_op, be2_op, rm2_op, rv2_op)
    w1f_m, b1f_m, w2f_m, b2f_m = _fold_bn(
        W1_mach, b1_mach, g1_mach, be1_mach, rm1_mach, rv1_mach,
        W2_mach, b2_mach, g2_mach, be2_mach, rm2_mach, rv2_mach)

    w1s = jnp.stack([w1f_op, w1f_m])
    b1s = jnp.stack([b1f_op, b1f_m])[:, None, :]
    w2s = jnp.stack([w2f_op, w2f_m])
    b2s = jnp.stack([b2f_op, b2f_m])[:, None, :]
    epss = jnp.stack([eps_om, eps_mo])

    out_mach, out_op = _tc_mlp(agg, x_mach, x_op, w1s, b1s, w2s, b2s, epss)
    return (out_op, out_mach)
